# baseline (speedup 1.0000x reference)
import jax
import jax.numpy as jnp
from jax import lax
from jax.experimental import pallas as pl
from jax.experimental.pallas import tpu as pltpu

_RR = 512


def _relayout_body(cb_ref, out_ref):
    out_ref[...] = cb_ref[...].reshape(out_ref.shape)


def _make_gather_body(T):
    def _gather_body(idx_ref, cb_ref, out_ref, scratch_ref):
        s = pl.program_id(1)

        @pl.when(s == 0)
        def _():
            def tok(i, carry):
                row8 = idx_ref[0, 0, i]
                scratch_ref[pl.ds(8 * i, 8), :] = cb_ref[pl.ds(row8, 8), :]
                return carry
            lax.fori_loop(0, T, tok, 0, unroll=64)

        slab = scratch_ref[pl.Slice(s, T, 8), :]
        out_ref[0, :, :] = slab.T

    return _gather_body


def kernel(indices, codebook):
    B, T = indices.shape
    V, D = codebook.shape
    idx = jnp.clip(indices.astype(jnp.int32), 0, V - 1) * 8
    idx = idx.reshape(B, 1, T)

    cb2 = pl.pallas_call(
        _relayout_body,
        grid=(V // _RR,),
        in_specs=[pl.BlockSpec((_RR, D), lambda r: (r, 0))],
        out_specs=pl.BlockSpec((_RR * 8, D // 8), lambda r: (r, 0)),
        out_shape=jax.ShapeDtypeStruct((V * 8, D // 8), jnp.float32),
    )(codebook)

    out = pl.pallas_call(
        _make_gather_body(T),
        grid=(B, D // 128),
        in_specs=[
            pl.BlockSpec((1, 1, T), lambda b, s: (b, 0, 0),
                         memory_space=pltpu.SMEM),
            pl.BlockSpec((V * 8, D // 8), lambda b, s: (0, 0)),
        ],
        out_specs=pl.BlockSpec((1, 128, T), lambda b, s: (b, s, 0)),
        out_shape=jax.ShapeDtypeStruct((B, D, T), jnp.float32),
        scratch_shapes=[pltpu.VMEM((T * 8, D // 8), jnp.float32)],
    )(idx, cb2)
    return out


# speedup vs baseline: 1.3962x; 1.3962x over previous
import jax
import jax.numpy as jnp
from jax import lax
from jax.experimental import pallas as pl
from jax.experimental.pallas import tpu as pltpu

_TT = 2048
_RR = 512


def _relayout_body(cb_ref, out_ref):
    out_ref[...] = cb_ref[...].reshape(out_ref.shape)


def _gather_body(idx_ref, cb_ref, out_ref, scratch_ref):
    def tok(i, carry):
        row8 = idx_ref[0, 0, i]
        scratch_ref[pl.ds(8 * i, 8), :] = cb_ref[pl.ds(row8, 8), :]
        return carry

    lax.fori_loop(0, _TT, tok, 0, unroll=64)
    for s in range(8):
        slab = scratch_ref[pl.Slice(s, _TT, 8), :]
        out_ref[0, pl.ds(128 * s, 128), :] = slab.T


def kernel(indices, codebook):
    B, T = indices.shape
    V, D = codebook.shape
    NT = T // _TT
    idx = jnp.clip(indices.astype(jnp.int32), 0, V - 1) * 8
    idx = idx.reshape(B * NT, 1, _TT)

    cb2 = pl.pallas_call(
        _relayout_body,
        grid=(V // _RR,),
        in_specs=[pl.BlockSpec((_RR, D), lambda r: (r, 0))],
        out_specs=pl.BlockSpec((_RR * 8, D // 8), lambda r: (r, 0)),
        out_shape=jax.ShapeDtypeStruct((V * 8, D // 8), jnp.float32),
    )(codebook)

    out = pl.pallas_call(
        _gather_body,
        grid=(B, NT),
        in_specs=[
            pl.BlockSpec((1, 1, _TT), lambda b, t: (b * NT + t, 0, 0),
                         memory_space=pltpu.SMEM),
            pl.BlockSpec((V * 8, D // 8), lambda b, t: (0, 0)),
        ],
        out_specs=pl.BlockSpec((1, D, _TT), lambda b, t: (b, 0, t)),
        out_shape=jax.ShapeDtypeStruct((B, D, T), jnp.float32),
        scratch_shapes=[pltpu.VMEM((_TT * 8, D // 8), jnp.float32)],
    )(idx, cb2)
    return out
